# baseline (speedup 1.0000x reference)
import functools

import jax
import jax.numpy as jnp
import numpy as np
from jax.experimental import pallas as pl

N = 512
D = 64
MSG = 64
F_HID = 128
K = 8
_E2 = np.zeros((K * N, 2 * K), np.float32)
for _k in range(K):
    _E2[_k * N:(_k + 1) * N, _k] = 1.0
    _E2[_k * N:(_k + 1) * N, K + _k] = 1.0


def _bf(v):
    return v.astype(jnp.bfloat16)


def _ab_body(hid_ref, wf1a_ref, bf1_ref, ahi_ref, alo_ref):
    a = jnp.dot(hid_ref[:], wf1a_ref[:], preferred_element_type=jnp.float32) + bf1_ref[:]
    ahi = a.astype(jnp.bfloat16)
    ahi_ref[:] = ahi
    alo_ref[:] = (a - ahi.astype(jnp.float32)).astype(jnp.bfloat16)


def _sum_body(lhs_ref, a2_ref, wf1bt_ref, wf2t_ref, adjt3_ref, s_ref):
    c = pl.program_id(0)

    @pl.when(c == 0)
    def _():
        s_ref[:] = jnp.zeros_like(s_ref)

    rhs = jnp.concatenate([a2_ref[0], wf1bt_ref[:]], axis=0)
    pre = jnp.dot(lhs_ref[:], rhs, preferred_element_type=jnp.float32)
    relu = jnp.maximum(pre, 0.0)
    msgs = jnp.dot(relu, wf2t_ref[:], preferred_element_type=jnp.float32)
    m = adjt3_ref[0]
    acc = s_ref[:]
    for k in range(K):
        acc = acc + m[:, k:k + 1] * msgs[k * N:(k + 1) * N, :]
    s_ref[:] = acc


def _tail_body(s_ref, adjt_ref, x_ref, hprev_ref, cprev_ref,
               bf2_ref, wiht_ref, whht_ref, bsum_ref,
               wo1t_ref, bo1_ref, wo2t_ref, bo2_ref,
               out_ref, h_ref, c_ref):
    deg = jnp.sum(adjt_ref[:], axis=1, keepdims=True)
    msg = s_ref[:] + deg * bf2_ref[:]

    inp = jnp.concatenate([x_ref[:], msg], axis=1).astype(jnp.bfloat16)
    gates = (jnp.dot(inp, wiht_ref[:], preferred_element_type=jnp.float32)
             + jnp.dot(hprev_ref[:], whht_ref[:], preferred_element_type=jnp.float32)
             + bsum_ref[:])
    i_g = jax.nn.sigmoid(gates[:, 0 * D:1 * D])
    f_g = jax.nn.sigmoid(gates[:, 1 * D:2 * D])
    g_g = jnp.tanh(gates[:, 2 * D:3 * D])
    o_g = jax.nn.sigmoid(gates[:, 3 * D:4 * D])
    c_new = f_g * cprev_ref[:] + i_g * g_g
    h_new = o_g * jnp.tanh(c_new)

    hid1 = jnp.maximum(
        jnp.dot(h_new.astype(jnp.bfloat16), wo1t_ref[:],
                preferred_element_type=jnp.float32) + bo1_ref[:], 0.0)
    out_ref[:] = (jnp.dot(hid1.astype(jnp.bfloat16), wo2t_ref[:],
                          preferred_element_type=jnp.float32) + bo2_ref[:])
    h_ref[:] = h_new
    c_ref[:] = c_new


@functools.partial(jax.jit, static_argnames=("interpret",))
def _run(adjacency_matrix, x, hidden, h_h, h_c, Wf1, bf1, Wf2, bf2,
         W_ih, W_hh, b_ih, b_hh, Wo1, bo1, Wo2, bo2, interpret=False):
    n, d = hidden.shape
    adjt = adjacency_matrix.T.astype(jnp.float32)
    adjt3 = adjt.reshape(n, n // K, K).transpose(1, 0, 2)
    hid16 = _bf(hidden)

    a_hi, a_lo = pl.pallas_call(
        _ab_body,
        out_shape=[jax.ShapeDtypeStruct((n, F_HID), jnp.bfloat16)] * 2,
        interpret=interpret,
    )(hid16, _bf(Wf1[:, :d].T), bf1[None, :])
    a2 = jnp.concatenate([a_hi.reshape(n // K, K, F_HID),
                          a_lo.reshape(n // K, K, F_HID)], axis=1)

    lhs = jnp.concatenate([jnp.asarray(_E2, jnp.bfloat16),
                           jnp.tile(hid16, (K, 1))], axis=1)

    S = pl.pallas_call(
        _sum_body,
        grid=(n // K,),
        in_specs=[
            pl.BlockSpec((K * n, 2 * K + d), lambda c: (0, 0)),
            pl.BlockSpec((1, 2 * K, F_HID), lambda c: (c, 0, 0)),
            pl.BlockSpec((d, F_HID), lambda c: (0, 0)),
            pl.BlockSpec((F_HID, MSG), lambda c: (0, 0)),
            pl.BlockSpec((1, n, K), lambda c: (c, 0, 0)),
        ],
        out_specs=pl.BlockSpec((n, MSG), lambda c: (0, 0)),
        out_shape=jax.ShapeDtypeStruct((n, MSG), jnp.float32),
        interpret=interpret,
    )(lhs, a2, _bf(Wf1[:, d:].T), _bf(Wf2.T).astype(jnp.float32), adjt3)

    out, h_new, c_new = pl.pallas_call(
        _tail_body,
        out_shape=[
            jax.ShapeDtypeStruct((n, Wo2.shape[0]), jnp.float32),
            jax.ShapeDtypeStruct((n, d), jnp.float32),
            jax.ShapeDtypeStruct((n, d), jnp.float32),
        ],
        interpret=interpret,
    )(S, adjt, x, _bf(h_h[0]), h_c[0],
      bf2[None, :], _bf(W_ih.T), _bf(W_hh.T), (b_ih + b_hh)[None, :],
      _bf(Wo1.T), bo1[None, :], _bf(Wo2.T), bo2[None, :])
    return out, h_new, h_new[None, :, :], c_new[None, :, :]


def kernel(adjacency_matrix, x, hidden, h_h, h_c, Wf1, bf1, Wf2, bf2,
           W_ih, W_hh, b_ih, b_hh, Wo1, bo1, Wo2, bo2):
    return _run(adjacency_matrix, x, hidden, h_h, h_c, Wf1, bf1, Wf2, bf2,
                W_ih, W_hh, b_ih, b_hh, Wo1, bo1, Wo2, bo2)


# speedup vs baseline: 1.3184x; 1.3184x over previous
import functools

import jax
import jax.numpy as jnp
import numpy as np
from jax.experimental import pallas as pl

N = 512
D = 64
MSG = 64
F_HID = 128
K = 8
_E2T = np.zeros((2 * K, K * N), np.float32)
for _k in range(K):
    _E2T[_k, _k * N:(_k + 1) * N] = 1.0
    _E2T[K + _k, _k * N:(_k + 1) * N] = 1.0


def _bf(v):
    return v.astype(jnp.bfloat16)


def _ab_body(hidt_ref, wf1a_ref, bf1_ref, ahi_ref, alo_ref):
    a = jnp.dot(wf1a_ref[:], hidt_ref[:], preferred_element_type=jnp.float32) + bf1_ref[:]
    ahi = a.astype(jnp.bfloat16)
    ahi_ref[:] = ahi
    alo_ref[:] = (a - ahi.astype(jnp.float32)).astype(jnp.bfloat16)


def _sum_body(lhst_ref, a2t_ref, wf1b_ref, wf2_ref, adj3_ref, s_ref):
    c = pl.program_id(0)

    @pl.when(c == 0)
    def _():
        s_ref[:] = jnp.zeros_like(s_ref)

    rhst = jnp.concatenate([a2t_ref[0], wf1b_ref[:]], axis=1)
    pre = jnp.dot(rhst, lhst_ref[:], preferred_element_type=jnp.float32)
    relu = jnp.maximum(pre, 0.0)
    msgs = jnp.dot(wf2_ref[:], relu, preferred_element_type=jnp.float32)
    m = adj3_ref[0]
    acc = s_ref[:]
    for k in range(K):
        acc = acc + m[k:k + 1, :] * msgs[:, k * N:(k + 1) * N]
    s_ref[:] = acc


def _tail_body(st_ref, adj_ref, xt_ref, hprevt_ref, cprevt_ref,
               bf2_ref, wih_ref, whh_ref, bsum_ref,
               wo1_ref, bo1_ref, wo2_ref, bo2_ref,
               outt_ref, ht_ref, ct_ref):
    deg = jnp.sum(adj_ref[:], axis=0, keepdims=True)
    msgt = st_ref[:] + deg * bf2_ref[:]

    inpt = jnp.concatenate([xt_ref[:], msgt], axis=0).astype(jnp.bfloat16)
    gates = (jnp.dot(wih_ref[:], inpt, preferred_element_type=jnp.float32)
             + jnp.dot(whh_ref[:], hprevt_ref[:], preferred_element_type=jnp.float32)
             + bsum_ref[:])
    i_g = jax.nn.sigmoid(gates[0 * D:1 * D, :])
    f_g = jax.nn.sigmoid(gates[1 * D:2 * D, :])
    g_g = jnp.tanh(gates[2 * D:3 * D, :])
    o_g = jax.nn.sigmoid(gates[3 * D:4 * D, :])
    c_new = f_g * cprevt_ref[:] + i_g * g_g
    h_new = o_g * jnp.tanh(c_new)

    hid1 = jnp.maximum(
        jnp.dot(wo1_ref[:], h_new.astype(jnp.bfloat16),
                preferred_element_type=jnp.float32) + bo1_ref[:], 0.0)
    outt_ref[:] = (jnp.dot(wo2_ref[:], hid1.astype(jnp.bfloat16),
                           preferred_element_type=jnp.float32) + bo2_ref[:])
    ht_ref[:] = h_new
    ct_ref[:] = c_new


@functools.partial(jax.jit, static_argnames=("interpret",))
def _run(adjacency_matrix, x, hidden, h_h, h_c, Wf1, bf1, Wf2, bf2,
         W_ih, W_hh, b_ih, b_hh, Wo1, bo1, Wo2, bo2, interpret=False):
    n, d = hidden.shape
    adjf = adjacency_matrix.astype(jnp.float32)
    adj3 = adjf.reshape(n // K, K, n)
    hidt16 = _bf(hidden.T)

    a_hi, a_lo = pl.pallas_call(
        _ab_body,
        out_shape=[jax.ShapeDtypeStruct((F_HID, n), jnp.bfloat16)] * 2,
        interpret=interpret,
    )(hidt16, _bf(Wf1[:, :d]), bf1[:, None])
    a2t = jnp.concatenate([
        a_hi.reshape(F_HID, n // K, K).transpose(1, 0, 2),
        a_lo.reshape(F_HID, n // K, K).transpose(1, 0, 2),
    ], axis=2)

    lhst = jnp.concatenate([jnp.asarray(_E2T, jnp.bfloat16),
                            jnp.tile(hidt16, (1, K))], axis=0)

    St = pl.pallas_call(
        _sum_body,
        grid=(n // K,),
        in_specs=[
            pl.BlockSpec((2 * K + d, K * n), lambda c: (0, 0)),
            pl.BlockSpec((1, F_HID, 2 * K), lambda c: (c, 0, 0)),
            pl.BlockSpec((F_HID, d), lambda c: (0, 0)),
            pl.BlockSpec((MSG, F_HID), lambda c: (0, 0)),
            pl.BlockSpec((1, K, n), lambda c: (c, 0, 0)),
        ],
        out_specs=pl.BlockSpec((MSG, n), lambda c: (0, 0)),
        out_shape=jax.ShapeDtypeStruct((MSG, n), jnp.float32),
        interpret=interpret,
    )(lhst, a2t, _bf(Wf1[:, d:]), _bf(Wf2).astype(jnp.float32), adj3)

    outt, h_t, c_t = pl.pallas_call(
        _tail_body,
        out_shape=[
            jax.ShapeDtypeStruct((Wo2.shape[0], n), jnp.float32),
            jax.ShapeDtypeStruct((d, n), jnp.float32),
            jax.ShapeDtypeStruct((d, n), jnp.float32),
        ],
        interpret=interpret,
    )(St, adjf, x.T, _bf(h_h[0].T), h_c[0].T,
      bf2[:, None], _bf(W_ih), _bf(W_hh), (b_ih + b_hh)[:, None],
      _bf(Wo1), bo1[:, None], _bf(Wo2), bo2[:, None])
    h_new = h_t.T
    return outt.T, h_new, h_new[None, :, :], c_t.T[None, :, :]


def kernel(adjacency_matrix, x, hidden, h_h, h_c, Wf1, bf1, Wf2, bf2,
           W_ih, W_hh, b_ih, b_hh, Wo1, bo1, Wo2, bo2):
    return _run(adjacency_matrix, x, hidden, h_h, h_c, Wf1, bf1, Wf2, bf2,
                W_ih, W_hh, b_ih, b_hh, Wo1, bo1, Wo2, bo2)
